# Initial kernel scaffold; baseline (speedup 1.0000x reference)
import functools

import jax
import jax.numpy as jnp
from jax import lax
from jax.experimental import pallas as pl
from jax.experimental.pallas import tpu as pltpu
from jax.experimental.pallas import tpu_sc as plsc

N = 10000
E = 160000
D = 512
DS = 128
NSLAB = D // DS
NC, NS = 2, 16
CHUNK = 128
NCHUNKS = E // CHUNK
CPC = NCHUNKS // NC
DEG_CPT = -(-CPC // NS)
AGG_CPT = -(-NCHUNKS // NS)
ZROWS = N // NS
BM = 1000


def _sc_degree(dst2d, ones_src, zeros_src):
    mesh = plsc.VectorSubcoreMesh(core_axis_name="c", subcore_axis_name="s")

    @functools.partial(
        pl.kernel,
        mesh=mesh,
        out_type=jax.ShapeDtypeStruct((NC, N, 16), jnp.float32),
        scratch_types=[
            pltpu.VMEM((CHUNK,), jnp.int32),
            pltpu.VMEM((CHUNK, 16), jnp.float32),
            pltpu.VMEM((ZROWS, 16), jnp.float32),
            pltpu.VMEM_SHARED((N, 16), jnp.float32),
            pltpu.SemaphoreType.DMA,
        ],
    )
    def k(dst_hbm, ones_hbm, zeros_hbm, out_hbm, idx_v, ones_v, zeros_v, acc_sh, sem):
        c = lax.axis_index("c")
        s = lax.axis_index("s")
        pltpu.sync_copy(ones_hbm, ones_v)
        pltpu.sync_copy(zeros_hbm, zeros_v)
        pltpu.sync_copy(zeros_v, acc_sh.at[pl.ds(s * ZROWS, ZROWS)])
        plsc.subcore_barrier()

        @pl.loop(0, DEG_CPT)
        def _(kk):
            lc = s + kk * NS

            @pl.when(lc < CPC)
            def _():
                pltpu.sync_copy(dst_hbm.at[c * CPC + lc], idx_v)
                pltpu.sync_copy(ones_v, acc_sh.at[idx_v], add=True)

        plsc.subcore_barrier()
        pltpu.sync_copy(
            acc_sh.at[pl.ds(s * ZROWS, ZROWS)],
            out_hbm.at[c, pl.ds(s * ZROWS, ZROWS)],
        )

    return k(dst2d, ones_src, zeros_src)


def _sc_aggregate(g4, src1d, dst2d, zeros_src):
    mesh = plsc.VectorSubcoreMesh(core_axis_name="c", subcore_axis_name="s")

    @functools.partial(
        pl.kernel,
        mesh=mesh,
        out_type=jax.ShapeDtypeStruct((NSLAB, N, DS), jnp.float32),
        scratch_types=[
            pltpu.VMEM((CHUNK,), jnp.int32),
            pltpu.VMEM((CHUNK,), jnp.int32),
            pltpu.VMEM((CHUNK, DS), jnp.float32),
            pltpu.VMEM((125, DS), jnp.float32),
            pltpu.VMEM_SHARED((N, DS), jnp.float32),
            pltpu.SemaphoreType.DMA,
        ],
    )
    def k(g_hbm, src_hbm, dst_hbm, z_hbm, out_hbm, si_v, di_v, rows_v, z_v, acc_sh, sem):
        c = lax.axis_index("c")
        s = lax.axis_index("s")
        pltpu.sync_copy(z_hbm, z_v)
        for p in range(NSLAB // NC):
            slab = c * (NSLAB // NC) + p

            @pl.loop(0, 5)
            def _(z):
                pltpu.sync_copy(z_v, acc_sh.at[pl.ds(s * ZROWS + z * 125, 125)])

            plsc.subcore_barrier()

            @pl.loop(0, AGG_CPT)
            def _(kk):
                lc = s + kk * NS

                @pl.when(lc < NCHUNKS)
                def _():
                    pltpu.sync_copy(src_hbm.at[pl.ds(lc * CHUNK, CHUNK)], si_v)
                    pltpu.sync_copy(dst_hbm.at[lc], di_v)
                    pltpu.async_copy(g_hbm.at[slab].at[si_v], rows_v, sem).wait()
                    pltpu.sync_copy(rows_v, acc_sh.at[di_v], add=True)

            plsc.subcore_barrier()
            pltpu.sync_copy(
                acc_sh.at[pl.ds(s * ZROWS, ZROWS)],
                out_hbm.at[slab, pl.ds(s * ZROWS, ZROWS)],
            )
            plsc.subcore_barrier()

    return k(g4, src1d, dst2d, zeros_src)


def _tc_transform(x, W, degp):
    def body(x_ref, w_ref, da_ref, db_ref, g_ref, slh_ref):
        h = jnp.dot(x_ref[...], w_ref[...], preferred_element_type=jnp.float32)
        deg = 1.0 + da_ref[0, :, 0] + db_ref[0, :, 0]
        dis = lax.rsqrt(deg)[:, None]
        g = h * dis
        slh_ref[...] = g * dis
        for p in range(NSLAB):
            g_ref[p, :, :] = g[:, p * DS:(p + 1) * DS]

    return pl.pallas_call(
        body,
        grid=(N // BM,),
        in_specs=[
            pl.BlockSpec((BM, D), lambda i: (i, 0)),
            pl.BlockSpec((D, D), lambda i: (0, 0)),
            pl.BlockSpec((1, BM, 16), lambda i: (0, i, 0)),
            pl.BlockSpec((1, BM, 16), lambda i: (1, i, 0)),
        ],
        out_specs=[
            pl.BlockSpec((NSLAB, BM, DS), lambda i: (0, i, 0)),
            pl.BlockSpec((BM, D), lambda i: (i, 0)),
        ],
        out_shape=[
            jax.ShapeDtypeStruct((NSLAB, N, DS), jnp.float32),
            jax.ShapeDtypeStruct((N, D), jnp.float32),
        ],
    )(x, W, degp, degp)


def _tc_combine(acc4, slh, degp, b_row):
    def body(a_ref, slh_ref, da_ref, db_ref, b_ref, o_ref):
        deg = 1.0 + da_ref[0, :, 0] + db_ref[0, :, 0]
        dis = lax.rsqrt(deg)[:, None]
        acc = jnp.concatenate([a_ref[p] for p in range(NSLAB)], axis=1)
        o_ref[...] = acc * dis + slh_ref[...] + b_ref[...]

    return pl.pallas_call(
        body,
        grid=(N // BM,),
        in_specs=[
            pl.BlockSpec((NSLAB, BM, DS), lambda i: (0, i, 0)),
            pl.BlockSpec((BM, D), lambda i: (i, 0)),
            pl.BlockSpec((1, BM, 16), lambda i: (0, i, 0)),
            pl.BlockSpec((1, BM, 16), lambda i: (1, i, 0)),
            pl.BlockSpec((1, D), lambda i: (0, 0)),
        ],
        out_specs=pl.BlockSpec((BM, D), lambda i: (i, 0)),
        out_shape=jax.ShapeDtypeStruct((N, D), jnp.float32),
    )(acc4, slh, degp, degp, b_row)


def kernel(x, edge_index, W, b):
    src = edge_index[0].astype(jnp.int32)
    dst = edge_index[1].astype(jnp.int32)
    dst2d = dst.reshape(NCHUNKS, CHUNK)
    ones_src = jnp.ones((CHUNK, 16), jnp.float32)
    zeros16 = jnp.zeros((ZROWS, 16), jnp.float32)
    zeros128 = jnp.zeros((125, DS), jnp.float32)

    degp = _sc_degree(dst2d, ones_src, zeros16)
    g4, slh = _tc_transform(x, W, degp)
    acc4 = _sc_aggregate(g4, src, dst2d, zeros128)
    return _tc_combine(acc4, slh, degp, b.reshape(1, D))


# speedup vs baseline: 8.6625x; 8.6625x over previous
import functools

import jax
import jax.numpy as jnp
from jax import lax
from jax.experimental import pallas as pl
from jax.experimental.pallas import tpu as pltpu
from jax.experimental.pallas import tpu_sc as plsc

N = 10000
E = 160000
D = 512
DS = 128
NSLAB = D // DS
NC, NS = 2, 16
CHUNK = 128
NCHUNKS = E // CHUNK
CPC = NCHUNKS // NC
DEG_CPT = -(-CPC // NS)
AGG_CPT = -(-NCHUNKS // NS)
ZROWS = N // NS
BM = 1000


def _sc_degree(dst1d, ones_src, zeros_src):
    mesh = plsc.VectorSubcoreMesh(core_axis_name="c", subcore_axis_name="s")

    @functools.partial(
        pl.kernel,
        mesh=mesh,
        out_type=jax.ShapeDtypeStruct((NC, NS, ZROWS, DS), jnp.float32),
        scratch_types=[
            pltpu.VMEM((CHUNK,), jnp.int32),
            pltpu.VMEM((CHUNK, DS), jnp.float32),
            pltpu.VMEM_SHARED((N, DS), jnp.float32),
        ],
    )
    def k(dst_hbm, ones_hbm, zeros_hbm, out_hbm, idx_v, ones_v, acc_sh):
        c = lax.axis_index("c")
        s = lax.axis_index("s")
        pltpu.sync_copy(ones_hbm, ones_v)
        pltpu.sync_copy(zeros_hbm, acc_sh.at[pl.ds(s * ZROWS, ZROWS)])
        plsc.subcore_barrier()

        @pl.loop(0, DEG_CPT)
        def _(kk):
            lc = s + kk * NS

            @pl.when(lc < CPC)
            def _():
                pltpu.sync_copy(
                    dst_hbm.at[pl.ds((c * CPC + lc) * CHUNK, CHUNK)], idx_v)
                pltpu.sync_copy(ones_v, acc_sh.at[idx_v], add=True)

        plsc.subcore_barrier()
        pltpu.sync_copy(acc_sh.at[pl.ds(s * ZROWS, ZROWS)], out_hbm.at[c, s])

    return k(dst1d, ones_src, zeros_src)


def _sc_aggregate(g4, src1d, dst1d, zeros_src):
    mesh = plsc.VectorSubcoreMesh(core_axis_name="c", subcore_axis_name="s")

    @functools.partial(
        pl.kernel,
        mesh=mesh,
        out_type=jax.ShapeDtypeStruct((NSLAB, NS, ZROWS, DS), jnp.float32),
        scratch_types=[
            pltpu.VMEM((CHUNK,), jnp.int32),
            pltpu.VMEM((CHUNK,), jnp.int32),
            pltpu.VMEM((CHUNK, DS), jnp.float32),
            pltpu.VMEM_SHARED((N, DS), jnp.float32),
            pltpu.SemaphoreType.DMA,
        ],
    )
    def k(g_hbm, src_hbm, dst_hbm, z_hbm, out_hbm, si_v, di_v, rows_v, acc_sh, sem):
        c = lax.axis_index("c")
        s = lax.axis_index("s")
        for p in range(NSLAB // NC):
            slab = c * (NSLAB // NC) + p
            pltpu.sync_copy(z_hbm, acc_sh.at[pl.ds(s * ZROWS, ZROWS)])
            plsc.subcore_barrier()

            @pl.loop(0, AGG_CPT)
            def _(kk):
                lc = s + kk * NS

                @pl.when(lc < NCHUNKS)
                def _():
                    pltpu.sync_copy(src_hbm.at[pl.ds(lc * CHUNK, CHUNK)], si_v)
                    pltpu.sync_copy(dst_hbm.at[pl.ds(lc * CHUNK, CHUNK)], di_v)
                    pltpu.async_copy(g_hbm.at[slab].at[si_v], rows_v, sem).wait()
                    pltpu.sync_copy(rows_v, acc_sh.at[di_v], add=True)

            plsc.subcore_barrier()
            pltpu.sync_copy(acc_sh.at[pl.ds(s * ZROWS, ZROWS)], out_hbm.at[slab, s])
            plsc.subcore_barrier()

    return k(g4, src1d, dst1d, zeros_src)


def _tc_transform(x, W, degp):
    def body(x_ref, w_ref, da_ref, db_ref, g_ref, slh_ref):
        h = jnp.dot(x_ref[...], w_ref[...], preferred_element_type=jnp.float32)
        deg = 1.0 + da_ref[0, :, 0] + db_ref[0, :, 0]
        dis = lax.rsqrt(deg)[:, None]
        g = h * dis
        slh_ref[...] = g * dis
        for p in range(NSLAB):
            g_ref[p, :, :] = g[:, p * DS:(p + 1) * DS]

    return pl.pallas_call(
        body,
        grid=(N // BM,),
        in_specs=[
            pl.BlockSpec((BM, D), lambda i: (i, 0)),
            pl.BlockSpec((D, D), lambda i: (0, 0)),
            pl.BlockSpec((1, BM, DS), lambda i: (0, i, 0)),
            pl.BlockSpec((1, BM, DS), lambda i: (1, i, 0)),
        ],
        out_specs=[
            pl.BlockSpec((NSLAB, BM, DS), lambda i: (0, i, 0)),
            pl.BlockSpec((BM, D), lambda i: (i, 0)),
        ],
        out_shape=[
            jax.ShapeDtypeStruct((NSLAB, N, DS), jnp.float32),
            jax.ShapeDtypeStruct((N, D), jnp.float32),
        ],
    )(x, W, degp, degp)


def _tc_combine(acc4, slh, degp, b_row):
    def body(a_ref, slh_ref, da_ref, db_ref, b_ref, o_ref):
        deg = 1.0 + da_ref[0, :, 0] + db_ref[0, :, 0]
        dis = lax.rsqrt(deg)[:, None]
        acc = jnp.concatenate([a_ref[p] for p in range(NSLAB)], axis=1)
        o_ref[...] = acc * dis + slh_ref[...] + b_ref[...]

    return pl.pallas_call(
        body,
        grid=(N // BM,),
        in_specs=[
            pl.BlockSpec((NSLAB, BM, DS), lambda i: (0, i, 0)),
            pl.BlockSpec((BM, D), lambda i: (i, 0)),
            pl.BlockSpec((1, BM, DS), lambda i: (0, i, 0)),
            pl.BlockSpec((1, BM, DS), lambda i: (1, i, 0)),
            pl.BlockSpec((1, D), lambda i: (0, 0)),
        ],
        out_specs=pl.BlockSpec((BM, D), lambda i: (i, 0)),
        out_shape=jax.ShapeDtypeStruct((N, D), jnp.float32),
    )(acc4, slh, degp, degp, b_row)


def kernel(x, edge_index, W, b):
    src = edge_index[0].astype(jnp.int32)
    dst = edge_index[1].astype(jnp.int32)
    ones_src = jnp.ones((CHUNK, DS), jnp.float32)
    zeros_rows = jnp.zeros((ZROWS, DS), jnp.float32)

    degp = _sc_degree(dst, ones_src, zeros_rows).reshape(NC, N, DS)
    g4, slh = _tc_transform(x, W, degp)
    acc4 = _sc_aggregate(g4, src, dst, zeros_rows).reshape(NSLAB, N, DS)
    out = _tc_combine(acc4, slh, degp, b[None, :])
    return out
